# Initial kernel scaffold; baseline (speedup 1.0000x reference)
import jax
import jax.numpy as jnp
from jax.experimental import pallas as pl


def kernel(feat, params, A, h1id, batch):
    raise NotImplementedError("write your pallas kernel here")


# speedup vs baseline: 2.0789x; 2.0789x over previous
import functools

import jax
import jax.numpy as jnp
from jax import lax
from jax.experimental import pallas as pl
from jax.experimental.pallas import tpu as pltpu
from jax.experimental.pallas import tpu_sc as plsc

_N = 10000
_E = 160000
_NC, _NS = 2, 16
_NW = _NC * _NS
_NPAD = 10112
_RPT = _NPAD // _NS
_CHUNK = 128
_NCHUNK = 40
_EPAD = _NW * _NCHUNK * _CHUNK
_GROWS = 64
_BM = 2000


def _sc_scatter_partials(xc, srcp, dstp):
  nchunks = xc.shape[0]
  f = xc.shape[2]
  mesh = plsc.VectorSubcoreMesh(core_axis_name="c", subcore_axis_name="s",
                                num_cores=_NC, num_subcores=_NS)

  @functools.partial(
      pl.kernel,
      out_type=jax.ShapeDtypeStruct((_NC, _NPAD, nchunks * f), jnp.float32),
      mesh=mesh,
      scratch_types=[
          pltpu.VMEM((_NCHUNK, _CHUNK), jnp.int32),
          pltpu.VMEM((_NCHUNK, _CHUNK), jnp.int32),
          pltpu.VMEM((_CHUNK, f), jnp.float32),
          pltpu.VMEM((_CHUNK, f), jnp.float32),
          pltpu.VMEM_SHARED((_NPAD, f), jnp.float32),
          pltpu.SemaphoreType.DMA,
          pltpu.SemaphoreType.DMA,
      ],
  )
  def k(x_hbm, src_hbm, dst_hbm, out_hbm, srcv, dstv, buf0, buf1, acc,
        sem0, sem1):
    cid = lax.axis_index("c")
    sid = lax.axis_index("s")
    wid = cid * _NS + sid
    base = sid * _RPT

    pltpu.sync_copy(src_hbm.at[wid], srcv)
    pltpu.sync_copy(dst_hbm.at[wid], dstv)

    for c in range(nchunks):
      def zrow(i, carry):
        for j in range(f // 16):
          buf0[i, pl.ds(j * 16, 16)] = jnp.zeros((16,), jnp.float32)
        return carry
      lax.fori_loop(0, _CHUNK, zrow, 0)
      for t in range(_RPT // _CHUNK):
        pltpu.sync_copy(buf0, acc.at[pl.ds(base + t * _CHUNK, _CHUNK)])
      rem = _RPT % _CHUNK
      if rem:
        pltpu.sync_copy(buf0.at[pl.ds(0, rem)],
                        acc.at[pl.ds(base + (_RPT // _CHUNK) * _CHUNK, rem)])
      plsc.subcore_barrier()

      xv = x_hbm.at[c]

      def body(p, carry):
        j0 = 2 * p
        c0 = pltpu.async_copy(xv.at[srcv.at[j0]], buf0, sem0)
        c1 = pltpu.async_copy(xv.at[srcv.at[j0 + 1]], buf1, sem1)
        c0.wait()
        pltpu.sync_copy(buf0, acc.at[dstv.at[j0]], add=True)
        c1.wait()
        pltpu.sync_copy(buf1, acc.at[dstv.at[j0 + 1]], add=True)
        return carry
      lax.fori_loop(0, _NCHUNK // 2, body, 0)

      plsc.subcore_barrier()
      pltpu.sync_copy(acc.at[pl.ds(base, _RPT)],
                      out_hbm.at[cid, pl.ds(base, _RPT), pl.ds(c * f, f)])
      if c + 1 < nchunks:
        plsc.subcore_barrier()

  return k(xc, srcp, dstp)


def _sc_gather_rows(h, rows):
  f = h.shape[1]
  mesh = plsc.VectorSubcoreMesh(core_axis_name="c", subcore_axis_name="s",
                                num_cores=_NC, num_subcores=_NS)

  @functools.partial(
      pl.kernel,
      out_type=jax.ShapeDtypeStruct((_NW, _GROWS, f), jnp.float32),
      mesh=mesh,
      scratch_types=[
          pltpu.VMEM((_GROWS,), jnp.int32),
          pltpu.VMEM((_GROWS, f), jnp.float32),
          pltpu.SemaphoreType.DMA,
      ],
  )
  def k(h_hbm, rows_hbm, out_hbm, idxv, buf, sem):
    cid = lax.axis_index("c")
    sid = lax.axis_index("s")
    wid = cid * _NS + sid
    pltpu.sync_copy(rows_hbm.at[wid], idxv)
    pltpu.async_copy(h_hbm.at[idxv], buf, sem).wait()
    pltpu.sync_copy(buf, out_hbm.at[wid])

  return k(h, rows)


def _tc_bn_relu(h2, stats, gamma, beta):
  n, fin = h2.shape

  def body(hr, sr, gr, br, outr):
    m = sr[0, :] / n
    var = sr[1, :] / n - m * m
    scale = gr[0, :] * lax.rsqrt(var + 1e-5)
    shift = br[0, :] - m * scale
    outr[...] = jnp.maximum(hr[...] * scale + shift, 0.0)

  return pl.pallas_call(
      body,
      grid=(n // _BM,),
      in_specs=[pl.BlockSpec((_BM, fin), lambda i: (i, 0)),
                pl.BlockSpec((2, fin), lambda i: (0, 0)),
                pl.BlockSpec((1, fin), lambda i: (0, 0)),
                pl.BlockSpec((1, fin), lambda i: (0, 0))],
      out_specs=pl.BlockSpec((_BM, fin), lambda i: (i, 0)),
      out_shape=jax.ShapeDtypeStruct((n, fin), jnp.float32),
  )(h2, stats, gamma, beta)


def _tc_gin_layer(x, parts, eps, wa, ba, wb, bb):
  n, fin = x.shape
  fmid = wa.shape[1]
  fout = wb.shape[1]

  def body(xr, pr, er, war, bar, wbr, bbr, outr, statr):
    i = pl.program_id(0)
    hp = (1.0 + er[0, 0]) * xr[...] + (pr[0] + pr[1])
    h = jnp.dot(hp, war[...], preferred_element_type=jnp.float32) + bar[0, :]
    h = jnp.maximum(h, 0.0)
    h2 = jnp.dot(h, wbr[...], preferred_element_type=jnp.float32) + bbr[0, :]
    outr[...] = h2
    s = jnp.concatenate([jnp.sum(h2, 0, keepdims=True),
                         jnp.sum(h2 * h2, 0, keepdims=True)], 0)

    @pl.when(i == 0)
    def _():
      statr[...] = s

    @pl.when(i > 0)
    def _():
      statr[...] = statr[...] + s

  return pl.pallas_call(
      body,
      grid=(n // _BM,),
      in_specs=[pl.BlockSpec((_BM, fin), lambda i: (i, 0)),
                pl.BlockSpec((2, _BM, fin), lambda i: (0, i, 0)),
                pl.BlockSpec((1, 1), lambda i: (0, 0)),
                pl.BlockSpec((fin, fmid), lambda i: (0, 0)),
                pl.BlockSpec((1, fmid), lambda i: (0, 0)),
                pl.BlockSpec((fmid, fout), lambda i: (0, 0)),
                pl.BlockSpec((1, fout), lambda i: (0, 0))],
      out_specs=[pl.BlockSpec((_BM, fout), lambda i: (i, 0)),
                 pl.BlockSpec((2, fout), lambda i: (0, 0))],
      out_shape=[jax.ShapeDtypeStruct((n, fout), jnp.float32),
                 jax.ShapeDtypeStruct((2, fout), jnp.float32)],
  )(x, parts, eps, wa, ba, wb, bb)


def _tc_head(ef, stats, gamma, beta, wc1, bc1, prelu, wc2, bc2):
  m_rows, fin = ef.shape
  fout = wc2.shape[1]

  def body(er, sr, gr, br, w1r, b1r, pr, w2r, b2r, outr):
    m = sr[0, :] / _N
    var = sr[1, :] / _N - m * m
    scale = gr[0, :] * lax.rsqrt(var + 1e-5)
    shift = br[0, :] - m * scale
    x = er[...] * scale + shift
    z = jnp.dot(x, w1r[...], preferred_element_type=jnp.float32) + b1r[0, :]
    z = jnp.where(z > 0, z, pr[0, :] * z)
    outr[...] = jnp.dot(z, w2r[...],
                        preferred_element_type=jnp.float32) + b2r[0, :]

  return pl.pallas_call(
      body,
      in_specs=[pl.BlockSpec(ef.shape, lambda: (0, 0)),
                pl.BlockSpec(stats.shape, lambda: (0, 0)),
                pl.BlockSpec(gamma.shape, lambda: (0, 0)),
                pl.BlockSpec(beta.shape, lambda: (0, 0)),
                pl.BlockSpec(wc1.shape, lambda: (0, 0)),
                pl.BlockSpec(bc1.shape, lambda: (0, 0)),
                pl.BlockSpec(prelu.shape, lambda: (0, 0)),
                pl.BlockSpec(wc2.shape, lambda: (0, 0)),
                pl.BlockSpec(bc2.shape, lambda: (0, 0))],
      out_specs=pl.BlockSpec((m_rows, fout), lambda: (0, 0)),
      out_shape=jax.ShapeDtypeStruct((m_rows, fout), jnp.float32),
  )(ef, stats, gamma, beta, wc1, bc1, prelu, wc2, bc2)


def kernel(feat, params, A, h1id, batch):
  del batch
  convs = [params['conv%d' % i] for i in range(4)]
  cls = params['cls']

  src, dst = A[0], A[1]
  pad = _EPAD - _E
  srcp = jnp.concatenate(
      [src, jnp.zeros((pad,), jnp.int32)]).reshape(_NW, _NCHUNK, _CHUNK)
  dstp = jnp.concatenate(
      [dst, jnp.full((pad,), _N, jnp.int32)]).reshape(_NW, _NCHUNK, _CHUNK)

  def b2(v):
    return v[None, :]

  c0 = convs[0]
  feat4 = jnp.swapaxes(feat.reshape(_N, 4, 128), 0, 1)
  parts = _sc_scatter_partials(feat4, srcp, dstp)
  h2, stats = _tc_gin_layer(feat, parts, c0['eps'].reshape(1, 1), c0['Wa'],
                            b2(c0['ba']), c0['Wb'], b2(c0['bb']))

  for i in (1, 2, 3):
    ci = convs[i]
    x = _tc_bn_relu(h2, stats, b2(convs[i - 1]['gamma']),
                    b2(convs[i - 1]['beta']))
    parts = _sc_scatter_partials(x[None], srcp, dstp)
    h2, stats = _tc_gin_layer(x, parts, ci['eps'].reshape(1, 1), ci['Wa'],
                              b2(ci['ba']), ci['Wb'], b2(ci['bb']))

  nb = _N // 100
  rows = jnp.repeat(jnp.arange(100, dtype=jnp.int32) * nb, 20) + h1id
  rows_pad = jnp.concatenate(
      [rows, jnp.zeros((_NW * _GROWS - rows.shape[0],), jnp.int32)]
  ).reshape(_NW, _GROWS)
  ef = _sc_gather_rows(h2, rows_pad).reshape(_NW * _GROWS, -1)[:rows.shape[0]]
  return _tc_head(ef, stats, b2(convs[3]['gamma']), b2(convs[3]['beta']),
                  cls['Wc1'], b2(cls['bc1']), b2(cls['prelu']), cls['Wc2'],
                  b2(cls['bc2']))
